# Initial kernel scaffold; baseline (speedup 1.0000x reference)
import jax
import jax.numpy as jnp
from jax import lax
from jax.experimental import pallas as pl
from jax.experimental.pallas import tpu as pltpu
from jax.experimental.pallas import tpu_sc as plsc

NUM_ROWS = 819200
D = 64
NC = 2
NS = 16
NW = NC * NS
B_PER_W = NUM_ROWS // NW
SUB = 128
GROUP = 512
SUBS_PER_GROUP = GROUP // SUB
N_CHUNKS = B_PER_W // SUB
N_GROUPS = B_PER_W // GROUP
NBUF = 2


def _sc_body(idx_hbm, table_hbm, out_hbm, idx_v, rows0, rows1,
             sg0, sg1, ss0, ss1):
  wid = lax.axis_index("s") * NC + lax.axis_index("c")
  base = wid * B_PER_W
  rows = (rows0, rows1)
  sg = (sg0, sg1)
  ss = (ss0, ss1)

  pltpu.sync_copy(idx_hbm.at[wid], idx_v)

  def fire_gathers(g, b):
    for k in range(SUBS_PER_GROUP):
      pltpu.async_copy(
          table_hbm.at[idx_v.at[g * SUBS_PER_GROUP + k]],
          rows[b].at[pl.ds(k * SUB, SUB)],
          sg[b])

  def wait_gathers(b):
    pltpu.make_async_copy(table_hbm.at[pl.ds(0, GROUP)], rows[b], sg[b]).wait()

  def fire_scatter(g, b):
    pltpu.async_copy(rows[b], out_hbm.at[pl.ds(base + g * GROUP, GROUP)],
                     ss[b])

  def wait_scatter(b):
    pltpu.make_async_copy(rows[b], out_hbm.at[pl.ds(base, GROUP)],
                          ss[b]).wait()

  for b in range(NBUF):
    fire_gathers(b, b)

  @pl.loop(0, N_GROUPS - NBUF, step=NBUF)
  def _(t):
    for b in range(NBUF):
      g = t + b
      wait_gathers(b)
      fire_scatter(g, b)
      wait_scatter(b)
      fire_gathers(g + NBUF, b)

  for b in range(NBUF):
    g = N_GROUPS - NBUF + b
    wait_gathers(b)
    fire_scatter(g, b)
  for b in range(NBUF):
    wait_scatter(b)


_lookup = pl.kernel(
    _sc_body,
    out_type=jax.ShapeDtypeStruct((NUM_ROWS, D), jnp.float32),
    mesh=plsc.VectorSubcoreMesh(core_axis_name="c", subcore_axis_name="s"),
    scratch_types=[
        pltpu.VMEM((N_CHUNKS, SUB), jnp.int32),
        pltpu.VMEM((GROUP, D), jnp.float32),
        pltpu.VMEM((GROUP, D), jnp.float32),
        pltpu.SemaphoreType.DMA,
        pltpu.SemaphoreType.DMA,
        pltpu.SemaphoreType.DMA,
        pltpu.SemaphoreType.DMA,
    ],
)


@jax.jit
def kernel(action_ids, embedding_weight):
  b, t = action_ids.shape
  idx = action_ids.reshape(NW, N_CHUNKS, SUB).astype(jnp.int32)
  out = _lookup(idx, embedding_weight)
  return out.reshape(b, t, D)


# speedup vs baseline: 1.8750x; 1.8750x over previous
import jax
import jax.numpy as jnp
from jax import lax
from jax.experimental import pallas as pl
from jax.experimental.pallas import tpu as pltpu
from jax.experimental.pallas import tpu_sc as plsc

NUM_ROWS = 819200
D = 64
NC = 2
NS = 16
NW = NC * NS
B_PER_W = NUM_ROWS // NW
SUB = 128
GROUP = 512
SUBS_PER_GROUP = GROUP // SUB
N_CHUNKS = B_PER_W // SUB
N_GROUPS = B_PER_W // GROUP
NBUF = 2


def _sc_body(idx_hbm, table_hbm, out_hbm, idx_v, rows0, rows1,
             sg0, sg1, ss0, ss1):
  wid = lax.axis_index("s") * NC + lax.axis_index("c")
  base = wid * B_PER_W
  rows = (rows0, rows1)
  sg = (sg0, sg1)
  ss = (ss0, ss1)

  pltpu.sync_copy(idx_hbm.at[wid], idx_v)

  def fire_gathers(g, b):
    for k in range(SUBS_PER_GROUP):
      pltpu.async_copy(
          table_hbm.at[idx_v.at[g * SUBS_PER_GROUP + k]],
          rows[b].at[pl.ds(k * SUB, SUB)],
          sg[b])

  def wait_gathers(b):
    pltpu.make_async_copy(table_hbm.at[pl.ds(0, GROUP)], rows[b], sg[b]).wait()

  def fire_scatter(g, b):
    pltpu.async_copy(rows[b], out_hbm.at[pl.ds(base + g * GROUP, GROUP)],
                     ss[b])

  def wait_scatter(b):
    pltpu.make_async_copy(rows[b], out_hbm.at[pl.ds(base, GROUP)],
                          ss[b]).wait()

  for b in range(NBUF):
    fire_gathers(b, b)

  @pl.loop(0, N_GROUPS - NBUF, step=NBUF)
  def _(t):
    for b in range(NBUF):
      g = t + b
      wait_gathers(b)
      fire_scatter(g, b)
      wait_scatter(b)
      fire_gathers(g + NBUF, b)

  for b in range(NBUF):
    g = N_GROUPS - NBUF + b
    wait_gathers(b)
    fire_scatter(g, b)
  for b in range(NBUF):
    wait_scatter(b)


_lookup = pl.kernel(
    _sc_body,
    out_type=jax.ShapeDtypeStruct((NUM_ROWS, D), jnp.float32),
    mesh=plsc.VectorSubcoreMesh(core_axis_name="c", subcore_axis_name="s"),
    scratch_types=[
        pltpu.VMEM((N_CHUNKS, SUB), jnp.int32),
        pltpu.VMEM((GROUP, D), jnp.float32),
        pltpu.VMEM((GROUP, D), jnp.float32),
        pltpu.SemaphoreType.DMA,
        pltpu.SemaphoreType.DMA,
        pltpu.SemaphoreType.DMA,
        pltpu.SemaphoreType.DMA,
    ],
    compiler_params=pltpu.CompilerParams(use_tc_tiling_on_sc=False),
)


@jax.jit
def kernel(action_ids, embedding_weight):
  b, t = action_ids.shape
  idx = action_ids.reshape(NW, N_CHUNKS, SUB).astype(jnp.int32)
  out = _lookup(idx, embedding_weight)
  return out.reshape(b, t, D)
